# Initial kernel scaffold; baseline (speedup 1.0000x reference)
import functools

import jax
import jax.numpy as jnp
from jax import lax
from jax.experimental import pallas as pl
from jax.experimental.pallas import tpu as pltpu
from jax.experimental.pallas import tpu_sc as plsc

N = 10000
E = 160000
HID = 256
HEADS = 4
C = 64
EDGE_DIM = 16
HALF = 128
NP = 10240
ROWW = 144
NSUB = 16
NCORE = 2
B = 80
EPW = E // NSUB
NCH = EPW // B
NROWS = NP // NSUB

BN = 2000
BE = 20000


def _fullspec(shape):
    return pl.BlockSpec(shape, lambda i: (0,) * len(shape))


def _pre_body(x_ref, wq, wk, wv, ws, bq, bk, bv, bs, g1, b1,
              qf, kf, vf, xr):
    x = x_ref[...]
    m = jnp.mean(x, axis=-1, keepdims=True)
    xc = x - m
    var = jnp.mean(xc * xc, axis=-1, keepdims=True)
    h = xc * lax.rsqrt(var + 1e-5) * g1[...] + b1[...]
    q = jnp.dot(h, wq[...], preferred_element_type=jnp.float32) + bq[...]
    k = jnp.dot(h, wk[...], preferred_element_type=jnp.float32) + bk[...]
    v = jnp.dot(h, wv[...], preferred_element_type=jnp.float32) + bv[...]
    qf[0], qf[1] = q[:, :HALF], q[:, HALF:]
    kf[0], kf[1] = k[:, :HALF], k[:, HALF:]
    vf[0], vf[1] = v[:, :HALF], v[:, HALF:]
    xr[...] = jnp.dot(h, ws[...], preferred_element_type=jnp.float32) + bs[...]


def _pre(x, wqT, wkT, wvT, wsT, bq, bk, bv, bs, g1, b1):
    split = pl.BlockSpec((NCORE, BN, HALF), lambda i: (0, i, 0))
    return pl.pallas_call(
        _pre_body,
        grid=(N // BN,),
        in_specs=[pl.BlockSpec((BN, HID), lambda i: (i, 0))]
        + [_fullspec((HID, HID))] * 4
        + [_fullspec((1, HID))] * 6,
        out_specs=[split, split, split,
                   pl.BlockSpec((BN, HID), lambda i: (i, 0))],
        out_shape=[jax.ShapeDtypeStruct((NCORE, N, HALF), jnp.float32)] * 3
        + [jax.ShapeDtypeStruct((N, HID), jnp.float32)],
    )(x, wqT, wkT, wvT, wsT, bq, bk, bv, bs, g1, b1)


def _eproj_body(ea_ref, we_ref, ef):
    e = jnp.dot(ea_ref[...], we_ref[...], preferred_element_type=jnp.float32)
    ef[0], ef[1] = e[:, :HALF], e[:, HALF:]


def _eproj(edge_attr, weT):
    return pl.pallas_call(
        _eproj_body,
        grid=(E // BE,),
        in_specs=[pl.BlockSpec((BE, EDGE_DIM), lambda i: (i, 0)),
                  _fullspec((EDGE_DIM, HID))],
        out_specs=pl.BlockSpec((NCORE, BE, HALF), lambda i: (0, i, 0)),
        out_shape=jax.ShapeDtypeStruct((NCORE, E, HALF), jnp.float32),
    )(edge_attr, weT)


def _sc_body(qf, kf, vf, ef, src, dst, out,
             acc, sidx, didx, didx2, qv, kv, vv, ev, msg,
             semq, semk, semv):
    c = lax.axis_index("c")
    s = lax.axis_index("s")
    cN = c * N

    def zrow(i, _):
        for r in range(ROWW // 16):
            msg[i, pl.ds(r * 16, 16)] = jnp.zeros((16,), jnp.float32)
        return 0
    lax.fori_loop(0, B, zrow, 0)
    for t in range(NROWS // B):
        pltpu.sync_copy(msg, acc.at[pl.ds(s * NROWS + t * B, B)])
    plsc.subcore_barrier()

    ii = lax.iota(jnp.int32, 16)
    oh0 = (ii == 0).astype(jnp.float32)
    oh1 = (ii == 1).astype(jnp.float32)

    def chunk(j, _):
        off = s * EPW + j * B
        pltpu.sync_copy(src.at[pl.ds(off, B)], sidx)
        pltpu.sync_copy(dst.at[pl.ds(off, B)], didx)
        for r in range(B // 16):
            sl = pl.ds(r * 16, 16)
            sidx[sl] = sidx[sl] + cN
            didx2[sl] = didx[sl] + cN
        cq = pltpu.async_copy(qf.at[didx2], qv, semq)
        ck = pltpu.async_copy(kf.at[sidx], kv, semk)
        cv = pltpu.async_copy(vf.at[sidx], vv, semv)
        pltpu.sync_copy(ef.at[pl.ds(c * E + off, B)], ev)
        cq.wait()
        ck.wait()
        cv.wait()

        def edge(i, _2):
            def head_logit(base):
                a = jnp.zeros((16,), jnp.float32)
                for r in range(4):
                    sl = pl.ds(base + r * 16, 16)
                    a = a + qv[i, sl] * (kv[i, sl] + ev[i, sl])
                return jnp.sum(a) * 0.125
            w0 = jnp.exp(jnp.full((16,), head_logit(0), jnp.float32))
            w1 = jnp.exp(jnp.full((16,), head_logit(64), jnp.float32))
            for r in range(8):
                sl = pl.ds(r * 16, 16)
                w = w0 if r < 4 else w1
                msg[i, sl] = w * (vv[i, sl] + ev[i, sl])
            msg[i, pl.ds(HALF, 16)] = w0 * oh0 + w1 * oh1
            return 0
        lax.fori_loop(0, B, edge, 0)
        pltpu.sync_copy(msg, acc.at[didx], add=True)
        return 0
    lax.fori_loop(0, NCH, chunk, 0)

    plsc.subcore_barrier()
    pltpu.sync_copy(acc.at[pl.ds(s * NROWS, NROWS)],
                    out.at[pl.ds(c * NP + s * NROWS, NROWS)])


def _sc_edge(qf, kf, vf, ef, src, dst):
    mesh = plsc.VectorSubcoreMesh(core_axis_name="c", subcore_axis_name="s")
    f = pl.kernel(
        _sc_body,
        out_type=jax.ShapeDtypeStruct((NCORE * NP, ROWW), jnp.float32),
        mesh=mesh,
        scratch_types=[
            pltpu.VMEM_SHARED((NP, ROWW), jnp.float32),
            pltpu.VMEM((B,), jnp.int32),
            pltpu.VMEM((B,), jnp.int32),
            pltpu.VMEM((B,), jnp.int32),
            pltpu.VMEM((B, HALF), jnp.float32),
            pltpu.VMEM((B, HALF), jnp.float32),
            pltpu.VMEM((B, HALF), jnp.float32),
            pltpu.VMEM((B, HALF), jnp.float32),
            pltpu.VMEM((B, ROWW), jnp.float32),
            pltpu.SemaphoreType.DMA,
            pltpu.SemaphoreType.DMA,
            pltpu.SemaphoreType.DMA,
        ],
    )
    return f(qf, kf, vf, ef, src, dst)


def _post_body(x_ref, of_ref, xr_ref, wba, wbb, g2, bg2, w1, b1, w2, b2,
               y_ref):
    x = x_ref[...]
    of = of_ref[...]
    eps = 1e-16
    o = jnp.concatenate(
        [of[0, :, 0:64] / (of[0, :, 128:129] + eps),
         of[0, :, 64:128] / (of[0, :, 129:130] + eps),
         of[1, :, 0:64] / (of[1, :, 128:129] + eps),
         of[1, :, 64:128] / (of[1, :, 129:130] + eps)], axis=-1)
    xr = xr_ref[...]
    z = (jnp.sum(o * wba[...], axis=-1, keepdims=True)
         + jnp.sum(xr * wbb[...], axis=-1, keepdims=True))
    beta = 1.0 / (1.0 + jnp.exp(-z))
    x1 = x + beta * xr + (1.0 - beta) * o
    m = jnp.mean(x1, axis=-1, keepdims=True)
    xc = x1 - m
    var = jnp.mean(xc * xc, axis=-1, keepdims=True)
    h2 = xc * lax.rsqrt(var + 1e-5) * g2[...] + bg2[...]
    mid = jnp.maximum(
        jnp.dot(h2, w1[...], preferred_element_type=jnp.float32) + b1[...],
        0.0)
    ff = jnp.dot(mid, w2[...], preferred_element_type=jnp.float32) + b2[...]
    y_ref[...] = x1 + ff


def _post(x, of, xr, wba, wbb, g2, bg2, w1T, b1, w2T, b2):
    return pl.pallas_call(
        _post_body,
        grid=(N // BN,),
        in_specs=[pl.BlockSpec((BN, HID), lambda i: (i, 0)),
                  pl.BlockSpec((NCORE, BN, ROWW), lambda i: (0, i, 0)),
                  pl.BlockSpec((BN, HID), lambda i: (i, 0)),
                  _fullspec((1, HID)), _fullspec((1, HID)),
                  _fullspec((1, HID)), _fullspec((1, HID)),
                  _fullspec((HID, 4 * HID)), _fullspec((1, 4 * HID)),
                  _fullspec((4 * HID, HID)), _fullspec((1, HID))],
        out_specs=pl.BlockSpec((BN, HID), lambda i: (i, 0)),
        out_shape=jax.ShapeDtypeStruct((N, HID), jnp.float32),
    )(x, of, xr, wba, wbb, g2, bg2, w1T, b1, w2T, b2)


def kernel(x, edge_index, edge_attr, params):
    p = params
    row = lambda a: a.reshape(1, -1)
    qf, kf, vf, xr = _pre(
        x, p["Wq"].T, p["Wk"].T, p["Wv"].T, p["Wskip"].T,
        row(p["bq"]), row(p["bk"]), row(p["bv"]), row(p["bskip"]),
        row(p["ln1_g"]), row(p["ln1_b"]))
    ef = _eproj(edge_attr, p["We"].T)
    of = _sc_edge(qf.reshape(NCORE * N, HALF), kf.reshape(NCORE * N, HALF),
                  vf.reshape(NCORE * N, HALF), ef.reshape(NCORE * E, HALF),
                  edge_index[0], edge_index[1])
    wb = p["Wbeta"][0]
    wba = row(wb[:HID] + wb[2 * HID:])
    wbb = row(wb[HID:2 * HID] - wb[2 * HID:])
    return _post(x, of.reshape(NCORE, NP, ROWW), xr, wba, wbb,
                 row(p["ln2_g"]), row(p["ln2_b"]),
                 p["W1"].T, row(p["b1"]), p["W2"].T, row(p["b2"]))


# speedup vs baseline: 11.2863x; 11.2863x over previous
import functools

import jax
import jax.numpy as jnp
from jax import lax
from jax.experimental import pallas as pl
from jax.experimental.pallas import tpu as pltpu
from jax.experimental.pallas import tpu_sc as plsc

N = 10000
E = 160000
HID = 256
HEADS = 4
C = 64
EDGE_DIM = 16
HALF = 128
NP = 10240
ROWW = 136
NSUB = 16
NCORE = 2
B = 40
EPW = E // NSUB
NCH = EPW // B
NROWS = NP // NSUB

BN = 2000
BE = 10000


def _fullspec(shape):
    return pl.BlockSpec(shape, lambda i: (0,) * len(shape))


def _pre_body(x_ref, wq, wk, wv, ws, bq, bk, bv, bs, g1, b1,
              qf, kf, vf, xr):
    x = x_ref[...]
    m = jnp.mean(x, axis=-1, keepdims=True)
    xc = x - m
    var = jnp.mean(xc * xc, axis=-1, keepdims=True)
    h = xc * lax.rsqrt(var + 1e-5) * g1[...] + b1[...]
    q = jnp.dot(h, wq[...], preferred_element_type=jnp.float32) + bq[...]
    k = jnp.dot(h, wk[...], preferred_element_type=jnp.float32) + bk[...]
    v = jnp.dot(h, wv[...], preferred_element_type=jnp.float32) + bv[...]
    qf[0], qf[1] = q[:, :HALF], q[:, HALF:]
    kf[0], kf[1] = k[:, :HALF], k[:, HALF:]
    vf[0], vf[1] = v[:, :HALF], v[:, HALF:]
    xr[...] = jnp.dot(h, ws[...], preferred_element_type=jnp.float32) + bs[...]


def _pre(x, wqT, wkT, wvT, wsT, bq, bk, bv, bs, g1, b1):
    split = pl.BlockSpec((NCORE, BN, HALF), lambda i: (0, i, 0))
    return pl.pallas_call(
        _pre_body,
        grid=(N // BN,),
        in_specs=[pl.BlockSpec((BN, HID), lambda i: (i, 0))]
        + [_fullspec((HID, HID))] * 4
        + [_fullspec((1, HID))] * 6,
        out_specs=[split, split, split,
                   pl.BlockSpec((BN, HID), lambda i: (i, 0))],
        out_shape=[jax.ShapeDtypeStruct((NCORE, N, HALF), jnp.float32)] * 3
        + [jax.ShapeDtypeStruct((N, HID), jnp.float32)],
    )(x, wqT, wkT, wvT, wsT, bq, bk, bv, bs, g1, b1)


def _eproj_body(ea_ref, we_ref, ei_ref, ef, s2, d2):
    e = jnp.dot(ea_ref[...], we_ref[...], preferred_element_type=jnp.float32)
    ef[0], ef[1] = e[:, :HALF], e[:, HALF:]
    ei = ei_ref[0]
    offs = lax.broadcasted_iota(jnp.int32, (NCORE, 1), 0) * N
    s2[0] = ei[0:1, :] + offs
    d2[0] = ei[1:2, :] + offs


def _eproj(edge_attr, weT, ei3):
    idx3 = pl.BlockSpec((1, NCORE, BE), lambda i: (i, 0, 0))
    return pl.pallas_call(
        _eproj_body,
        grid=(E // BE,),
        in_specs=[pl.BlockSpec((BE, EDGE_DIM), lambda i: (i, 0)),
                  _fullspec((EDGE_DIM, HID)), idx3],
        out_specs=[pl.BlockSpec((NCORE, BE, HALF), lambda i: (0, i, 0)),
                   idx3, idx3],
        out_shape=[jax.ShapeDtypeStruct((NCORE, E, HALF), jnp.float32),
                   jax.ShapeDtypeStruct((E // BE, NCORE, BE), jnp.int32),
                   jax.ShapeDtypeStruct((E // BE, NCORE, BE), jnp.int32)],
    )(edge_attr, weT, ei3)


def _sc_body(qf, kf, vf, ef, src2, dst2, out,
             acc, sidx, didx, didx2, qv, kv, vv, ev, msg,
             semq, semk, semv):
    c = lax.axis_index("c")
    s = lax.axis_index("s")

    ii = lax.iota(jnp.int32, 16)
    zv = jnp.zeros((16,), jnp.float32)
    tail_mask = ii < ROWW - HALF
    tail_cols = HALF + (ii % (ROWW - HALF))

    def zrow(i, _):
        for r in range(HALF // 16):
            msg[i, pl.ds(r * 16, 16)] = zv
        plsc.store_scatter(msg, [jnp.full((16,), i, jnp.int32), tail_cols],
                           zv, mask=tail_mask)
        return 0
    lax.fori_loop(0, B, zrow, 0)
    for t in range(NROWS // B):
        pltpu.sync_copy(msg, acc.at[pl.ds(s * NROWS + t * B, B)])
    plsc.subcore_barrier()

    oh0 = (ii == 0).astype(jnp.float32)
    oh1 = (ii == 1).astype(jnp.float32)
    den_mask = ii < 2

    @pl.loop(0, NCH)
    def chunk(j):
        off = s * EPW + j * B
        joff = j * B
        pltpu.sync_copy(src2.at[s, c, pl.ds(joff, B)], sidx)
        pltpu.sync_copy(dst2.at[s, 0, pl.ds(joff, B)], didx)
        pltpu.sync_copy(dst2.at[s, c, pl.ds(joff, B)], didx2)
        cq = pltpu.async_copy(qf.at[didx2], qv, semq)
        ck = pltpu.async_copy(kf.at[sidx], kv, semk)
        cv = pltpu.async_copy(vf.at[sidx], vv, semv)
        pltpu.sync_copy(ef.at[pl.ds(c * E + off, B)], ev)
        cq.wait()
        ck.wait()
        cv.wait()

        def edge(i, _2):
            def head_logit(base):
                a = jnp.zeros((16,), jnp.float32)
                for r in range(4):
                    sl = pl.ds(base + r * 16, 16)
                    a = a + qv[i, sl] * (kv[i, sl] + ev[i, sl])
                return jnp.sum(a) * 0.125
            w0 = jnp.exp(jnp.full((16,), head_logit(0), jnp.float32))
            w1 = jnp.exp(jnp.full((16,), head_logit(64), jnp.float32))
            for r in range(8):
                sl = pl.ds(r * 16, 16)
                w = w0 if r < 4 else w1
                msg[i, sl] = w * (vv[i, sl] + ev[i, sl])
            plsc.store_scatter(
                msg, [jnp.full((16,), i, jnp.int32), tail_cols],
                w0 * oh0 + w1 * oh1, mask=den_mask)
            return 0
        lax.fori_loop(0, B, edge, 0)
        pltpu.sync_copy(msg, acc.at[didx], add=True)

    plsc.subcore_barrier()
    pltpu.sync_copy(acc.at[pl.ds(s * NROWS, NROWS)],
                    out.at[pl.ds(c * NP + s * NROWS, NROWS)])


def _sc_edge(qf, kf, vf, ef, src2, dst2):
    mesh = plsc.VectorSubcoreMesh(core_axis_name="c", subcore_axis_name="s")
    f = pl.kernel(
        _sc_body,
        out_type=jax.ShapeDtypeStruct((NCORE * NP, ROWW), jnp.float32),
        mesh=mesh,
        compiler_params=pltpu.CompilerParams(needs_layout_passes=False,
                                             use_tc_tiling_on_sc=False),
        scratch_types=[
            pltpu.VMEM_SHARED((NP, ROWW), jnp.float32),
            pltpu.VMEM((B,), jnp.int32),
            pltpu.VMEM((B,), jnp.int32),
            pltpu.VMEM((B,), jnp.int32),
            pltpu.VMEM((B, HALF), jnp.float32),
            pltpu.VMEM((B, HALF), jnp.float32),
            pltpu.VMEM((B, HALF), jnp.float32),
            pltpu.VMEM((B, HALF), jnp.float32),
            pltpu.VMEM((B, ROWW), jnp.float32),
            pltpu.SemaphoreType.DMA,
            pltpu.SemaphoreType.DMA,
            pltpu.SemaphoreType.DMA,
        ],
    )
    return f(qf, kf, vf, ef, src2, dst2)


def _post_body(x_ref, of_ref, xr_ref, wba, wbb, g2, bg2, w1, b1, w2, b2,
               y_ref):
    x = x_ref[...]
    of = of_ref[...]
    eps = 1e-16
    o = jnp.concatenate(
        [of[0, :, 0:64] / (of[0, :, 128:129] + eps),
         of[0, :, 64:128] / (of[0, :, 129:130] + eps),
         of[1, :, 0:64] / (of[1, :, 128:129] + eps),
         of[1, :, 64:128] / (of[1, :, 129:130] + eps)], axis=-1)
    xr = xr_ref[...]
    z = (jnp.sum(o * wba[...], axis=-1, keepdims=True)
         + jnp.sum(xr * wbb[...], axis=-1, keepdims=True))
    beta = 1.0 / (1.0 + jnp.exp(-z))
    x1 = x + beta * xr + (1.0 - beta) * o
    m = jnp.mean(x1, axis=-1, keepdims=True)
    xc = x1 - m
    var = jnp.mean(xc * xc, axis=-1, keepdims=True)
    h2 = xc * lax.rsqrt(var + 1e-5) * g2[...] + bg2[...]
    mid = jnp.maximum(
        jnp.dot(h2, w1[...], preferred_element_type=jnp.float32) + b1[...],
        0.0)
    ff = jnp.dot(mid, w2[...], preferred_element_type=jnp.float32) + b2[...]
    y_ref[...] = x1 + ff


def _post(x, of, xr, wba, wbb, g2, bg2, w1T, b1, w2T, b2):
    return pl.pallas_call(
        _post_body,
        grid=(N // BN,),
        in_specs=[pl.BlockSpec((BN, HID), lambda i: (i, 0)),
                  pl.BlockSpec((NCORE, BN, ROWW), lambda i: (0, i, 0)),
                  pl.BlockSpec((BN, HID), lambda i: (i, 0)),
                  _fullspec((1, HID)), _fullspec((1, HID)),
                  _fullspec((1, HID)), _fullspec((1, HID)),
                  _fullspec((HID, 4 * HID)), _fullspec((1, 4 * HID)),
                  _fullspec((4 * HID, HID)), _fullspec((1, HID))],
        out_specs=pl.BlockSpec((BN, HID), lambda i: (i, 0)),
        out_shape=jax.ShapeDtypeStruct((N, HID), jnp.float32),
    )(x, of, xr, wba, wbb, g2, bg2, w1T, b1, w2T, b2)


def kernel(x, edge_index, edge_attr, params):
    p = params
    row = lambda a: a.reshape(1, -1)
    qf, kf, vf, xr = _pre(
        x, p["Wq"].T, p["Wk"].T, p["Wv"].T, p["Wskip"].T,
        row(p["bq"]), row(p["bk"]), row(p["bv"]), row(p["bskip"]),
        row(p["ln1_g"]), row(p["ln1_b"]))
    ei3 = edge_index.reshape(2, E // BE, BE).transpose(1, 0, 2)
    ef, src2, dst2 = _eproj(edge_attr, p["We"].T, ei3)
    of = _sc_edge(qf.reshape(NCORE * N, HALF), kf.reshape(NCORE * N, HALF),
                  vf.reshape(NCORE * N, HALF), ef.reshape(NCORE * E, HALF),
                  src2, dst2)
    wb = p["Wbeta"][0]
    wba = row(wb[:HID] + wb[2 * HID:])
    wbb = row(wb[HID:2 * HID] - wb[2 * HID:])
    return _post(x, of.reshape(NCORE, NP, ROWW), xr, wba, wbb,
                 row(p["ln2_g"]), row(p["ln2_b"]),
                 p["W1"].T, row(p["b1"]), p["W2"].T, row(p["b2"]))


# speedup vs baseline: 11.6053x; 1.0283x over previous
import functools

import jax
import jax.numpy as jnp
from jax import lax
from jax.experimental import pallas as pl
from jax.experimental.pallas import tpu as pltpu
from jax.experimental.pallas import tpu_sc as plsc

N = 10000
E = 160000
HID = 256
HEADS = 4
C = 64
EDGE_DIM = 16
HALF = 128
NP = 10240
ROWW = 136
NSUB = 16
NCORE = 2
B = 16
EPW = E // NSUB
NCH = EPW // B
NROWS = NP // NSUB

BN = 2000
BE = 10000


def _fullspec(shape):
    return pl.BlockSpec(shape, lambda i: (0,) * len(shape))


def _pre_body(x_ref, wq, wk, wv, ws, bq, bk, bv, bs, g1, b1,
              qf, kf, vf, xr):
    x = x_ref[...]
    m = jnp.mean(x, axis=-1, keepdims=True)
    xc = x - m
    var = jnp.mean(xc * xc, axis=-1, keepdims=True)
    h = xc * lax.rsqrt(var + 1e-5) * g1[...] + b1[...]
    q = jnp.dot(h, wq[...], preferred_element_type=jnp.float32) + bq[...]
    k = jnp.dot(h, wk[...], preferred_element_type=jnp.float32) + bk[...]
    v = jnp.dot(h, wv[...], preferred_element_type=jnp.float32) + bv[...]
    qf[0], qf[1] = q[:, :HALF], q[:, HALF:]
    kf[0], kf[1] = k[:, :HALF], k[:, HALF:]
    vf[0], vf[1] = v[:, :HALF], v[:, HALF:]
    xr[...] = jnp.dot(h, ws[...], preferred_element_type=jnp.float32) + bs[...]


def _pre(x, wqT, wkT, wvT, wsT, bq, bk, bv, bs, g1, b1):
    split = pl.BlockSpec((NCORE, BN, HALF), lambda i: (0, i, 0))
    return pl.pallas_call(
        _pre_body,
        grid=(N // BN,),
        in_specs=[pl.BlockSpec((BN, HID), lambda i: (i, 0))]
        + [_fullspec((HID, HID))] * 4
        + [_fullspec((1, HID))] * 6,
        out_specs=[split, split, split,
                   pl.BlockSpec((BN, HID), lambda i: (i, 0))],
        out_shape=[jax.ShapeDtypeStruct((NCORE, N, HALF), jnp.float32)] * 3
        + [jax.ShapeDtypeStruct((N, HID), jnp.float32)],
    )(x, wqT, wkT, wvT, wsT, bq, bk, bv, bs, g1, b1)


def _eproj_body(ea_ref, we_ref, ef):
    e = jnp.dot(ea_ref[...], we_ref[...], preferred_element_type=jnp.float32)
    ef[0], ef[1] = e[:, :HALF], e[:, HALF:]


def _eproj(edge_attr, weT):
    return pl.pallas_call(
        _eproj_body,
        grid=(E // BE,),
        in_specs=[pl.BlockSpec((BE, EDGE_DIM), lambda i: (i, 0)),
                  _fullspec((EDGE_DIM, HID))],
        out_specs=pl.BlockSpec((NCORE, BE, HALF), lambda i: (0, i, 0)),
        out_shape=jax.ShapeDtypeStruct((NCORE, E, HALF), jnp.float32),
    )(edge_attr, weT)


def _sc_body(qf, kf, vf, ef, idx, out, acc,
             idx3a, qva, kva, vva, eva, msga,
             idx3b, qvb, kvb, vvb, evb, msgb,
             sqa, ska, swa, sea, sqb, skb, swb, seb):
    c = lax.axis_index("c")
    s = lax.axis_index("s")
    seta = (idx3a, qva, kva, vva, eva, msga, sqa, ska, swa, sea)
    setb = (idx3b, qvb, kvb, vvb, evb, msgb, sqb, skb, swb, seb)

    ii = lax.iota(jnp.int32, 16)
    zv = jnp.zeros((16,), jnp.float32)
    tail_mask = ii < ROWW - HALF
    tail_cols = HALF + (ii % (ROWW - HALF))
    for i in range(B):
        for r in range(HALF // 16):
            msga[i, pl.ds(r * 16, 16)] = zv
        plsc.store_scatter(msga, [jnp.full((16,), i, jnp.int32), tail_cols],
                           zv, mask=tail_mask)
    for t in range(NROWS // B):
        pltpu.sync_copy(msga, acc.at[pl.ds(s * NROWS + t * B, B)])
    plsc.subcore_barrier()

    oh0 = (ii == 0).astype(jnp.float32)
    oh1 = (ii == 1).astype(jnp.float32)
    den_mask = ii < 2

    def load_set(t, S):
        idx3, qv, kv, vv, ev, _msg, sq, sk, sw, se = S
        pltpu.sync_copy(idx.at[c, s, t], idx3)
        svec = idx3[0]
        dvec2 = idx3[2]
        pltpu.async_copy(qf.at[dvec2], qv, sq)
        pltpu.async_copy(kf.at[svec], kv, sk)
        pltpu.async_copy(vf.at[svec], vv, sw)
        pltpu.async_copy(ef.at[pl.ds(c * E + s * EPW + t * B, B)], ev, se)

    def wait_set(S):
        idx3, qv, kv, vv, ev, _msg, sq, sk, sw, se = S
        pltpu.make_async_copy(qf.at[idx3[2]], qv, sq).wait()
        pltpu.make_async_copy(kf.at[idx3[0]], kv, sk).wait()
        pltpu.make_async_copy(vf.at[idx3[0]], vv, sw).wait()
        pltpu.make_async_copy(ef.at[pl.ds(0, B)], ev, se).wait()

    def compute_scatter(S):
        idx3, qv, kv, vv, ev, msg, *_ = S
        for i in range(B):
            def head_logit(base):
                a = jnp.zeros((16,), jnp.float32)
                for r in range(4):
                    sl = pl.ds(base + r * 16, 16)
                    a = a + qv[i, sl] * (kv[i, sl] + ev[i, sl])
                return jnp.sum(a) * 0.125
            w0 = jnp.exp(jnp.full((16,), head_logit(0), jnp.float32))
            w1 = jnp.exp(jnp.full((16,), head_logit(64), jnp.float32))
            for r in range(8):
                sl = pl.ds(r * 16, 16)
                w = w0 if r < 4 else w1
                msg[i, sl] = w * (vv[i, sl] + ev[i, sl])
            plsc.store_scatter(
                msg, [jnp.full((16,), i, jnp.int32), tail_cols],
                w0 * oh0 + w1 * oh1, mask=den_mask)
        pltpu.sync_copy(msg, acc.at[idx3[1]], add=True)

    def phase(t, cur, nxt, prefetch):
        if prefetch:
            load_set(t + 1, nxt)
        wait_set(cur)
        compute_scatter(cur)

    load_set(0, seta)

    @pl.loop(0, (NCH - 1) // 2)
    def lp(jp):
        t0 = jp * 2
        phase(t0, seta, setb, True)
        phase(t0 + 1, setb, seta, True)

    phase(NCH - 1, seta, setb, False)

    plsc.subcore_barrier()
    pltpu.sync_copy(acc.at[pl.ds(s * NROWS, NROWS)],
                    out.at[pl.ds(c * NP + s * NROWS, NROWS)])


def _sc_edge(qf, kf, vf, ef, idx):
    mesh = plsc.VectorSubcoreMesh(core_axis_name="c", subcore_axis_name="s")
    dbuf = [
        pltpu.VMEM((3, B), jnp.int32),
        pltpu.VMEM((B, HALF), jnp.float32),
        pltpu.VMEM((B, HALF), jnp.float32),
        pltpu.VMEM((B, HALF), jnp.float32),
        pltpu.VMEM((B, HALF), jnp.float32),
        pltpu.VMEM((B, ROWW), jnp.float32),
    ]
    f = pl.kernel(
        _sc_body,
        out_type=jax.ShapeDtypeStruct((NCORE * NP, ROWW), jnp.float32),
        mesh=mesh,
        compiler_params=pltpu.CompilerParams(needs_layout_passes=False,
                                             use_tc_tiling_on_sc=False),
        scratch_types=[pltpu.VMEM_SHARED((NP, ROWW), jnp.float32)]
        + dbuf + dbuf + [pltpu.SemaphoreType.DMA] * 8,
    )
    return f(qf, kf, vf, ef, idx)


def _post_body(x_ref, of_ref, xr_ref, wba, wbb, g2, bg2, w1, b1, w2, b2,
               y_ref):
    x = x_ref[...]
    of = of_ref[...]
    eps = 1e-16
    o = jnp.concatenate(
        [of[0, :, 0:64] / (of[0, :, 128:129] + eps),
         of[0, :, 64:128] / (of[0, :, 129:130] + eps),
         of[1, :, 0:64] / (of[1, :, 128:129] + eps),
         of[1, :, 64:128] / (of[1, :, 129:130] + eps)], axis=-1)
    xr = xr_ref[...]
    z = (jnp.sum(o * wba[...], axis=-1, keepdims=True)
         + jnp.sum(xr * wbb[...], axis=-1, keepdims=True))
    beta = 1.0 / (1.0 + jnp.exp(-z))
    x1 = x + beta * xr + (1.0 - beta) * o
    m = jnp.mean(x1, axis=-1, keepdims=True)
    xc = x1 - m
    var = jnp.mean(xc * xc, axis=-1, keepdims=True)
    h2 = xc * lax.rsqrt(var + 1e-5) * g2[...] + bg2[...]
    mid = jnp.maximum(
        jnp.dot(h2, w1[...], preferred_element_type=jnp.float32) + b1[...],
        0.0)
    ff = jnp.dot(mid, w2[...], preferred_element_type=jnp.float32) + b2[...]
    y_ref[...] = x1 + ff


def _post(x, of, xr, wba, wbb, g2, bg2, w1T, b1, w2T, b2):
    return pl.pallas_call(
        _post_body,
        grid=(N // BN,),
        in_specs=[pl.BlockSpec((BN, HID), lambda i: (i, 0)),
                  pl.BlockSpec((NCORE, BN, ROWW), lambda i: (0, i, 0)),
                  pl.BlockSpec((BN, HID), lambda i: (i, 0)),
                  _fullspec((1, HID)), _fullspec((1, HID)),
                  _fullspec((1, HID)), _fullspec((1, HID)),
                  _fullspec((HID, 4 * HID)), _fullspec((1, 4 * HID)),
                  _fullspec((4 * HID, HID)), _fullspec((1, HID))],
        out_specs=pl.BlockSpec((BN, HID), lambda i: (i, 0)),
        out_shape=jax.ShapeDtypeStruct((N, HID), jnp.float32),
    )(x, of, xr, wba, wbb, g2, bg2, w1T, b1, w2T, b2)


def kernel(x, edge_index, edge_attr, params):
    p = params
    row = lambda a: a.reshape(1, -1)
    qf, kf, vf, xr = _pre(
        x, p["Wq"].T, p["Wk"].T, p["Wv"].T, p["Wskip"].T,
        row(p["bq"]), row(p["bk"]), row(p["bv"]), row(p["bskip"]),
        row(p["ln1_g"]), row(p["ln1_b"]))
    ef = _eproj(edge_attr, p["We"].T)
    srcv, dstv = edge_index[0], edge_index[1]
    idx_all = jnp.stack([
        jnp.stack([srcv + cc * N, dstv, dstv + cc * N])
        .reshape(3, NSUB, NCH, B).transpose(1, 2, 0, 3)
        for cc in range(NCORE)])
    of = _sc_edge(qf.reshape(NCORE * N, HALF), kf.reshape(NCORE * N, HALF),
                  vf.reshape(NCORE * N, HALF), ef.reshape(NCORE * E, HALF),
                  idx_all)
    wb = p["Wbeta"][0]
    wba = row(wb[:HID] + wb[2 * HID:])
    wbb = row(wb[HID:2 * HID] - wb[2 * HID:])
    return _post(x, of.reshape(NCORE, NP, ROWW), xr, wba, wbb,
                 row(p["ln2_g"]), row(p["ln2_b"]),
                 p["W1"].T, row(p["b1"]), p["W2"].T, row(p["b2"]))


# speedup vs baseline: 12.7581x; 1.0993x over previous
import functools

import jax
import jax.numpy as jnp
from jax import lax
from jax.experimental import pallas as pl
from jax.experimental.pallas import tpu as pltpu
from jax.experimental.pallas import tpu_sc as plsc

N = 10000
E = 160000
HID = 256
HEADS = 4
C = 64
EDGE_DIM = 16
HALF = 128
NP = 10240
ROWW = 136
NSUB = 16
NCORE = 2
B = 16
EPW = E // NSUB
NCH = EPW // B
NROWS = NP // NSUB

BN = 2000
BE = 10000


def _fullspec(shape):
    return pl.BlockSpec(shape, lambda i: (0,) * len(shape))


def _pre_body(x_ref, wq, wk, wv, ws, bq, bk, bv, bs, g1, b1,
              qf, kf, vf, xr):
    x = x_ref[...]
    m = jnp.mean(x, axis=-1, keepdims=True)
    xc = x - m
    var = jnp.mean(xc * xc, axis=-1, keepdims=True)
    h = xc * lax.rsqrt(var + 1e-5) * g1[...] + b1[...]
    q = (jnp.dot(h, wq[...], preferred_element_type=jnp.float32)
         + bq[...]) * 0.125
    k = jnp.dot(h, wk[...], preferred_element_type=jnp.float32) + bk[...]
    v = jnp.dot(h, wv[...], preferred_element_type=jnp.float32) + bv[...]
    qf[0], qf[1] = q[:, :HALF], q[:, HALF:]
    kf[0], kf[1] = k[:, :HALF], k[:, HALF:]
    vf[0], vf[1] = v[:, :HALF], v[:, HALF:]
    xr[...] = jnp.dot(h, ws[...], preferred_element_type=jnp.float32) + bs[...]


def _pre(x, wqT, wkT, wvT, wsT, bq, bk, bv, bs, g1, b1):
    split = pl.BlockSpec((NCORE, BN, HALF), lambda i: (0, i, 0))
    return pl.pallas_call(
        _pre_body,
        grid=(N // BN,),
        in_specs=[pl.BlockSpec((BN, HID), lambda i: (i, 0))]
        + [_fullspec((HID, HID))] * 4
        + [_fullspec((1, HID))] * 6,
        out_specs=[split, split, split,
                   pl.BlockSpec((BN, HID), lambda i: (i, 0))],
        out_shape=[jax.ShapeDtypeStruct((NCORE, N, HALF), jnp.float32)] * 3
        + [jax.ShapeDtypeStruct((N, HID), jnp.float32)],
    )(x, wqT, wkT, wvT, wsT, bq, bk, bv, bs, g1, b1)


def _eproj_body(ea_ref, we_ref, ef):
    e = jnp.dot(ea_ref[...], we_ref[...], preferred_element_type=jnp.float32)
    ef[0], ef[1] = e[:, :HALF], e[:, HALF:]


def _eproj(edge_attr, weT):
    return pl.pallas_call(
        _eproj_body,
        grid=(E // BE,),
        in_specs=[pl.BlockSpec((BE, EDGE_DIM), lambda i: (i, 0)),
                  _fullspec((EDGE_DIM, HID))],
        out_specs=pl.BlockSpec((NCORE, BE, HALF), lambda i: (0, i, 0)),
        out_shape=jax.ShapeDtypeStruct((NCORE, E, HALF), jnp.float32),
    )(edge_attr, weT)


def _sc_body(qf, kf, vf, ef, idx, out, acc,
             idx3a, qva, kva, vva, eva, msga,
             idx3b, qvb, kvb, vvb, evb, msgb,
             sqa, ska, swa, sea, sqb, skb, swb, seb):
    c = lax.axis_index("c")
    s = lax.axis_index("s")
    seta = (idx3a, qva, kva, vva, eva, msga, sqa, ska, swa, sea)
    setb = (idx3b, qvb, kvb, vvb, evb, msgb, sqb, skb, swb, seb)

    ii = lax.iota(jnp.int32, 16)
    zv = jnp.zeros((16,), jnp.float32)
    tail_mask = ii < ROWW - HALF
    tail_cols = HALF + (ii % (ROWW - HALF))
    for i in range(B):
        for r in range(HALF // 16):
            msga[i, pl.ds(r * 16, 16)] = zv
        plsc.store_scatter(msga, [jnp.full((16,), i, jnp.int32), tail_cols],
                           zv, mask=tail_mask)
    for t in range(NROWS // B):
        pltpu.sync_copy(msga, acc.at[pl.ds(s * NROWS + t * B, B)])
    plsc.subcore_barrier()

    oh0 = (ii == 0).astype(jnp.float32)
    oh1 = (ii == 1).astype(jnp.float32)
    den_mask = ii < 2

    def load_set(t, S):
        idx3, qv, kv, vv, ev, _msg, sq, sk, sw, se = S
        pltpu.sync_copy(idx.at[c, s, t], idx3)
        svec = idx3[0]
        dvec2 = idx3[2]
        pltpu.async_copy(qf.at[dvec2], qv, sq)
        pltpu.async_copy(kf.at[svec], kv, sk)
        pltpu.async_copy(vf.at[svec], vv, sw)
        pltpu.async_copy(ef.at[pl.ds(c * E + s * EPW + t * B, B)], ev, se)

    def wait_set(S):
        idx3, qv, kv, vv, ev, _msg, sq, sk, sw, se = S
        pltpu.make_async_copy(qf.at[idx3[2]], qv, sq).wait()
        pltpu.make_async_copy(kf.at[idx3[0]], kv, sk).wait()
        pltpu.make_async_copy(vf.at[idx3[0]], vv, sw).wait()
        pltpu.make_async_copy(ef.at[pl.ds(0, B)], ev, se).wait()

    def compute_scatter(S):
        idx3, qv, kv, vv, ev, msg, *_ = S
        lane15 = jnp.full((16,), 15, jnp.int32)
        for i in range(B):
            evs = [ev[i, pl.ds(r * 16, 16)] for r in range(8)]

            def head_w(base):
                a = jnp.zeros((16,), jnp.float32)
                for r in range(4):
                    sl = pl.ds(base * 16 + r * 16, 16)
                    a = a + qv[i, sl] * (kv[i, sl] + evs[base + r])
                tot = jnp.take_along_axis(plsc.cumsum(a), lane15, axis=0,
                                          mode="promise_in_bounds")
                return jnp.exp(tot)
            w0 = head_w(0)
            w1 = head_w(4)
            for r in range(8):
                sl = pl.ds(r * 16, 16)
                w = w0 if r < 4 else w1
                msg[i, sl] = w * (vv[i, sl] + evs[r])
            plsc.store_scatter(
                msg, [jnp.full((16,), i, jnp.int32), tail_cols],
                w0 * oh0 + w1 * oh1, mask=den_mask)
        pltpu.sync_copy(msg, acc.at[idx3[1]], add=True)

    def phase(t, cur, nxt, prefetch):
        if prefetch:
            load_set(t + 1, nxt)
        wait_set(cur)
        compute_scatter(cur)

    load_set(0, seta)

    @pl.loop(0, (NCH - 1) // 2)
    def lp(jp):
        t0 = jp * 2
        phase(t0, seta, setb, True)
        phase(t0 + 1, setb, seta, True)

    phase(NCH - 1, seta, setb, False)

    plsc.subcore_barrier()
    pltpu.sync_copy(acc.at[pl.ds(s * NROWS, NROWS)],
                    out.at[pl.ds(c * NP + s * NROWS, NROWS)])


def _sc_edge(qf, kf, vf, ef, idx):
    mesh = plsc.VectorSubcoreMesh(core_axis_name="c", subcore_axis_name="s")
    dbuf = [
        pltpu.VMEM((3, B), jnp.int32),
        pltpu.VMEM((B, HALF), jnp.float32),
        pltpu.VMEM((B, HALF), jnp.float32),
        pltpu.VMEM((B, HALF), jnp.float32),
        pltpu.VMEM((B, HALF), jnp.float32),
        pltpu.VMEM((B, ROWW), jnp.float32),
    ]
    f = pl.kernel(
        _sc_body,
        out_type=jax.ShapeDtypeStruct((NCORE * NP, ROWW), jnp.float32),
        mesh=mesh,
        compiler_params=pltpu.CompilerParams(needs_layout_passes=False,
                                             use_tc_tiling_on_sc=False),
        scratch_types=[pltpu.VMEM_SHARED((NP, ROWW), jnp.float32)]
        + dbuf + dbuf + [pltpu.SemaphoreType.DMA] * 8,
    )
    return f(qf, kf, vf, ef, idx)


def _post_body(x_ref, of_ref, xr_ref, wba, wbb, g2, bg2, w1, b1, w2, b2,
               y_ref):
    x = x_ref[...]
    of = of_ref[...]
    eps = 1e-16
    o = jnp.concatenate(
        [of[0, :, 0:64] / (of[0, :, 128:129] + eps),
         of[0, :, 64:128] / (of[0, :, 129:130] + eps),
         of[1, :, 0:64] / (of[1, :, 128:129] + eps),
         of[1, :, 64:128] / (of[1, :, 129:130] + eps)], axis=-1)
    xr = xr_ref[...]
    z = (jnp.sum(o * wba[...], axis=-1, keepdims=True)
         + jnp.sum(xr * wbb[...], axis=-1, keepdims=True))
    beta = 1.0 / (1.0 + jnp.exp(-z))
    x1 = x + beta * xr + (1.0 - beta) * o
    m = jnp.mean(x1, axis=-1, keepdims=True)
    xc = x1 - m
    var = jnp.mean(xc * xc, axis=-1, keepdims=True)
    h2 = xc * lax.rsqrt(var + 1e-5) * g2[...] + bg2[...]
    mid = jnp.maximum(
        jnp.dot(h2, w1[...], preferred_element_type=jnp.float32) + b1[...],
        0.0)
    ff = jnp.dot(mid, w2[...], preferred_element_type=jnp.float32) + b2[...]
    y_ref[...] = x1 + ff


def _post(x, of, xr, wba, wbb, g2, bg2, w1T, b1, w2T, b2):
    return pl.pallas_call(
        _post_body,
        grid=(N // BN,),
        in_specs=[pl.BlockSpec((BN, HID), lambda i: (i, 0)),
                  pl.BlockSpec((NCORE, BN, ROWW), lambda i: (0, i, 0)),
                  pl.BlockSpec((BN, HID), lambda i: (i, 0)),
                  _fullspec((1, HID)), _fullspec((1, HID)),
                  _fullspec((1, HID)), _fullspec((1, HID)),
                  _fullspec((HID, 4 * HID)), _fullspec((1, 4 * HID)),
                  _fullspec((4 * HID, HID)), _fullspec((1, HID))],
        out_specs=pl.BlockSpec((BN, HID), lambda i: (i, 0)),
        out_shape=jax.ShapeDtypeStruct((N, HID), jnp.float32),
    )(x, of, xr, wba, wbb, g2, bg2, w1T, b1, w2T, b2)


def kernel(x, edge_index, edge_attr, params):
    p = params
    row = lambda a: a.reshape(1, -1)
    qf, kf, vf, xr = _pre(
        x, p["Wq"].T, p["Wk"].T, p["Wv"].T, p["Wskip"].T,
        row(p["bq"]), row(p["bk"]), row(p["bv"]), row(p["bskip"]),
        row(p["ln1_g"]), row(p["ln1_b"]))
    ef = _eproj(edge_attr, p["We"].T)
    srcv, dstv = edge_index[0], edge_index[1]
    idx_all = jnp.stack([
        jnp.stack([srcv + cc * N, dstv, dstv + cc * N])
        .reshape(3, NSUB, NCH, B).transpose(1, 2, 0, 3)
        for cc in range(NCORE)])
    of = _sc_edge(qf.reshape(NCORE * N, HALF), kf.reshape(NCORE * N, HALF),
                  vf.reshape(NCORE * N, HALF), ef.reshape(NCORE * E, HALF),
                  idx_all)
    wb = p["Wbeta"][0]
    wba = row(wb[:HID] + wb[2 * HID:])
    wbb = row(wb[HID:2 * HID] - wb[2 * HID:])
    return _post(x, of.reshape(NCORE, NP, ROWW), xr, wba, wbb,
                 row(p["ln2_g"]), row(p["ln2_b"]),
                 p["W1"].T, row(p["b1"]), p["W2"].T, row(p["b2"]))
